# baseline (speedup 1.0000x reference)
import functools

import jax
import jax.numpy as jnp
from jax import lax
from jax.experimental import pallas as pl
from jax.experimental.pallas import tpu as pltpu
from jax.experimental.pallas import tpu_sc as plsc

N = 10000
E = 320000
D = 128
NC = 2
NS = 16
NW = NC * NS
C = 64
NCH = 160
IW = 32
NWIN = NCH // IW
EPAD = NW * NCH * C
NPAD = 10240
RPT = NPAD // NS
TB = 1000
GRID = N // TB

F32 = jnp.float32
_mesh = plsc.VectorSubcoreMesh(core_axis_name="c", subcore_axis_name="s",
                               num_cores=NC, num_subcores=NS)
_HIGH = jax.lax.Precision.HIGHEST


def _dot_t(a, b):
    return lax.dot_general(a, b, (((1,), (1,)), ((), ())),
                           precision=_HIGH, preferred_element_type=F32)


def _dot_cc0(a, b):
    return lax.dot_general(a, b, (((0,), (0,)), ((), ())),
                           precision=_HIGH, preferred_element_type=F32)


@functools.partial(
    pl.kernel,
    out_type=jax.ShapeDtypeStruct((NC, NPAD), F32),
    mesh=_mesh,
    scratch_types=[
        pltpu.VMEM((NCH, C), jnp.int32),
        pltpu.VMEM((NCH, C), F32),
        pltpu.VMEM((RPT,), F32),
        pltpu.VMEM_SHARED((NPAD,), F32),
        pltpu.SemaphoreType.DMA,
    ],
)
def _sc_sumlines(cols_hbm, w_hbm, out_hbm, cols_v, w_v, zb, acc, ssem):
    c = lax.axis_index("c")
    s = lax.axis_index("s")
    wid = c * NS + s
    for i in range(RPT // 16):
        zb[pl.ds(i * 16, 16)] = jnp.zeros((16,), F32)
    pltpu.sync_copy(zb, acc.at[pl.ds(s * RPT, RPT)])
    pltpu.sync_copy(cols_hbm.at[wid], cols_v)
    pltpu.sync_copy(w_hbm.at[wid], w_v)
    plsc.subcore_barrier()

    def fire(j, carry):
        pltpu.sync_copy(w_v.at[j], acc.at[cols_v.at[j]], add=True)
        return carry

    lax.fori_loop(0, NCH, fire, 0)
    plsc.subcore_barrier()
    pltpu.sync_copy(acc.at[pl.ds(s * RPT, RPT)],
                    out_hbm.at[c, pl.ds(s * RPT, RPT)])


def _make_sc_spmm():
    @functools.partial(
        pl.kernel,
        out_type=jax.ShapeDtypeStruct((NC, NPAD, D), F32),
        mesh=_mesh,
        scratch_types=[
            pltpu.VMEM((IW, C), jnp.int32),
            pltpu.VMEM((IW, C), jnp.int32),
            pltpu.VMEM((IW // 2, 128), F32),
            pltpu.VMEM((C, D), F32),
            pltpu.VMEM((C, D), F32),
            pltpu.VMEM((C, D), F32),
            pltpu.VMEM((C, D), F32),
            pltpu.VMEM_SHARED((NPAD, D), F32),
            pltpu.SemaphoreType.DMA,
            pltpu.SemaphoreType.DMA,
            pltpu.SemaphoreType.DMA,
            pltpu.SemaphoreType.DMA,
            pltpu.SemaphoreType.DMA,
            pltpu.SemaphoreType.DMA,
            pltpu.SemaphoreType.DMA,
            pltpu.SemaphoreType.DMA,
        ],
    )
    def spmm(x_hbm, rows_hbm, cols_hbm, w_hbm, out_hbm,
             rw, cw, ww, gb, gb1, gb2, gb3, acc,
             gsem, gsem1, gsem2, gsem3, ssem, ssem1, ssem2, ssem3):
        c = lax.axis_index("c")
        s = lax.axis_index("s")
        wid = c * NS + s

        def zrow(r, carry):
            for v in range(8):
                gb[r, pl.ds(v * 16, 16)] = jnp.zeros((16,), F32)
            return carry

        lax.fori_loop(0, C, zrow, 0)
        for t in range(RPT // C):
            pltpu.sync_copy(gb, acc.at[pl.ds(s * RPT + t * C, C)])
        plsc.subcore_barrier()

        gbs = [gb, gb1, gb2, gb3]
        gss = [gsem, gsem1, gsem2, gsem3]
        sss = [ssem, ssem1, ssem2, ssem3]

        def win_body(w):
            pltpu.sync_copy(rows_hbm.at[wid, pl.ds(w * IW, IW)], rw)
            pltpu.sync_copy(cols_hbm.at[wid, pl.ds(w * IW, IW)], cw)
            wrows = IW * C // 128
            pltpu.sync_copy(w_hbm.at[wid, pl.ds(w * wrows, wrows)], ww)
            pltpu.async_copy(x_hbm.at[cw.at[0]], gbs[0], gss[0])

            def quad(t4, carry):
                for u in range(4):
                    t = t4 * 4 + u
                    b = u
                    bp = (u + 2) % 4
                    bn = (u + 1) % 4

                    @pl.when(t >= 2)
                    def _(t=t, bp=bp):
                        pltpu.make_async_copy(gbs[bp], acc.at[rw.at[t - 2]],
                                              sss[bp]).wait()

                    @pl.when(t + 1 < IW)
                    def _(t=t, bn=bn):
                        pltpu.async_copy(x_hbm.at[cw.at[t + 1]], gbs[bn],
                                         gss[bn])

                    pltpu.make_async_copy(x_hbm.at[cw.at[t]], gbs[b],
                                          gss[b]).wait()
                    th = t4 * 2 + u // 2

                    def sgrp(g, carry2, b=b, th=th, off=(u % 2) * C):
                        wvec = ww[th, pl.ds(off + g * 16, 16)]
                        for l in range(16):
                            wv = jnp.full((16,), wvec[l], F32)
                            e = g * 16 + l
                            for v in range(8):
                                sl = pl.ds(v * 16, 16)
                                gbs[b][e, sl] = gbs[b][e, sl] * wv
                        return carry2

                    lax.fori_loop(0, C // 16, sgrp, 0, unroll=2)

                    @pl.when(t < IW)
                    def _(t=t, b=b):
                        pltpu.async_copy(gbs[b], acc.at[rw.at[t]], sss[b],
                                         add=True)
                return carry

            lax.fori_loop(0, IW // 4, quad, 0)
            for t in (IW - 2, IW - 1):
                pltpu.make_async_copy(gbs[t % 4], acc.at[rw.at[t]],
                                      sss[t % 4]).wait()

        for w in range(NWIN):
            win_body(w)
        plsc.subcore_barrier()
        pltpu.sync_copy(acc.at[pl.ds(s * RPT, RPT)],
                        out_hbm.at[c, pl.ds(s * RPT, RPT)])

    return spmm


_sc_spmm = _make_sc_spmm()


def _prep_body(slp_ref, x_ref, sl_ref, cs_ref, ws_ref):
    i = pl.program_id(0)
    sl = slp_ref[0] + slp_ref[1]
    sl_ref[...] = sl

    @pl.when(i == 0)
    def _():
        cs_ref[...] = jnp.zeros((1, D), F32)
        ws_ref[...] = jnp.zeros((1, 1), F32)

    cs_ref[...] += jnp.sum(x_ref[...], axis=0, keepdims=True)
    ws_ref[...] += jnp.sum(sl).reshape(1, 1)


def _prep(slp, x):
    return pl.pallas_call(
        _prep_body,
        grid=(GRID,),
        in_specs=[
            pl.BlockSpec((NC, TB, 1), lambda i: (0, i, 0)),
            pl.BlockSpec((TB, D), lambda i: (i, 0)),
        ],
        out_specs=[
            pl.BlockSpec((TB, 1), lambda i: (i, 0)),
            pl.BlockSpec((1, D), lambda i: (0, 0)),
            pl.BlockSpec((1, 1), lambda i: (0, 0)),
        ],
        out_shape=[
            jax.ShapeDtypeStruct((N, 1), F32),
            jax.ShapeDtypeStruct((1, D), F32),
            jax.ShapeDtypeStruct((1, 1), F32),
        ],
    )(slp, x)


def _repy_body(cs_ref, ws_ref, w1_ref, b1_ref, w2_ref, b2_ref,
               lnw_ref, lnb_ref, y_ref, avg_ref):
    wsum = ws_ref[0, 0]
    avg = cs_ref[...] / wsum
    t = _dot_t(avg, w1_ref[...]) + b1_ref[...]
    t = jnp.maximum(t, 0.0)
    mu = jnp.mean(t, axis=1, keepdims=True)
    var = jnp.mean((t - mu) ** 2, axis=1, keepdims=True)
    t = (t - mu) / jnp.sqrt(var + 1e-5) * lnw_ref[...] + lnb_ref[...]
    t = _dot_t(t, w2_ref[...]) + b2_ref[...]
    y_ref[...] = 0.25 + 2.0 * jax.nn.sigmoid(t)
    avg_ref[...] = avg


def _repy(cs, ws, w1, b1, w2, b2, lnw, lnb):
    spec1d = pl.BlockSpec((1, D), lambda: (0, 0))
    return pl.pallas_call(
        _repy_body,
        in_specs=[spec1d, pl.BlockSpec((1, 1), lambda: (0, 0)),
                  pl.BlockSpec((D, D), lambda: (0, 0)), spec1d,
                  pl.BlockSpec((D, D), lambda: (0, 0)), spec1d,
                  spec1d, spec1d],
        out_specs=[spec1d, spec1d],
        out_shape=[jax.ShapeDtypeStruct((1, D), F32),
                   jax.ShapeDtypeStruct((1, D), F32)],
    )(cs, ws, w1, b1, w2, b2, lnw, lnb)


def _repx_body(x_ref, y_ref, xo_ref):
    xo_ref[...] = x_ref[...] * y_ref[...]


def _repx(x, y):
    return pl.pallas_call(
        _repx_body,
        grid=(GRID,),
        in_specs=[pl.BlockSpec((TB, D), lambda i: (i, 0)),
                  pl.BlockSpec((1, D), lambda i: (0, 0))],
        out_specs=pl.BlockSpec((TB, D), lambda i: (i, 0)),
        out_shape=jax.ShapeDtypeStruct((NPAD, D), F32),
    )(x, y)


def _repcov_body(x_ref, sl_ref, ws_ref, avg_ref, y_ref, kprev_ref,
                 k_ref, cov_s):
    i = pl.program_id(0)
    wsum = ws_ref[0, 0]

    @pl.when(i == 0)
    def _():
        cov_s[...] = jnp.zeros((D, D), F32)

    xb = x_ref[...]
    slb = sl_ref[...]
    d = jnp.sqrt(slb / wsum) * (xb / slb - avg_ref[...])
    cov_s[...] += _dot_cc0(d, d)

    @pl.when(i == GRID - 1)
    def _():
        cov = cov_s[...]
        q = y_ref[...]
        Q = _dot_cc0(q, q)
        eye = (lax.broadcasted_iota(jnp.int32, (D, D), 0)
               == lax.broadcasted_iota(jnp.int32, (D, D), 1)).astype(F32)
        t1 = jnp.sum(eye * cov * Q)
        t2 = jnp.sum(cov * Q)
        loss = t1 - (t2 - t1) / D
        c1 = jnp.sum(eye * cov)
        c2 = jnp.sum(cov)
        lscov = c1 - (c2 - c1) / D
        qq = jnp.sum(q * q)
        k_ref[...] = (kprev_ref[0, 0]
                      + loss / lscov * jnp.float32(D) / qq).reshape(1, 1)


def _repcov(x, sl, ws, avg, y, kprev):
    return pl.pallas_call(
        _repcov_body,
        grid=(GRID,),
        in_specs=[
            pl.BlockSpec((TB, D), lambda i: (i, 0)),
            pl.BlockSpec((TB, 1), lambda i: (i, 0)),
            pl.BlockSpec((1, 1), lambda i: (0, 0)),
            pl.BlockSpec((1, D), lambda i: (0, 0)),
            pl.BlockSpec((1, D), lambda i: (0, 0)),
            pl.BlockSpec((1, 1), lambda i: (0, 0)),
        ],
        out_specs=[pl.BlockSpec((1, 1), lambda i: (0, 0))],
        out_shape=[jax.ShapeDtypeStruct((1, 1), F32)],
        scratch_shapes=[pltpu.VMEM((D, D), F32)],
    )(x, sl, ws, avg, y, kprev)[0]


def _addp_body(p_ref, o_ref):
    o_ref[...] = p_ref[0] + p_ref[1]


def _addp(p):
    blk = NPAD // GRID
    return pl.pallas_call(
        _addp_body,
        grid=(GRID,),
        in_specs=[pl.BlockSpec((NC, blk, D), lambda i: (0, i, 0))],
        out_specs=pl.BlockSpec((blk, D), lambda i: (i, 0)),
        out_shape=jax.ShapeDtypeStruct((NPAD, D), F32),
    )(p)


def _make_tag(act):
    def body(x_ref, s1_ref, p2_ref, w0_ref, w1_ref, w2_ref, b_ref,
             xo_ref, cs_ref):
        i = pl.program_id(0)
        s2 = p2_ref[0] + p2_ref[1]
        o = (_dot_t(x_ref[...], w0_ref[...])
             + _dot_t(s1_ref[...], w1_ref[...])
             + _dot_t(s2, w2_ref[...])
             + b_ref[...])
        if act:
            o = jnp.where(o >= 0, o, 0.01 * o)
        xo_ref[...] = o

        @pl.when(i == 0)
        def _():
            cs_ref[...] = jnp.zeros((1, D), F32)

        cs_ref[...] += jnp.sum(o, axis=0, keepdims=True)

    def run(x, s1, p2, w0, w1, w2, b):
        return pl.pallas_call(
            body,
            grid=(GRID,),
            in_specs=[
                pl.BlockSpec((TB, D), lambda i: (i, 0)),
                pl.BlockSpec((TB, D), lambda i: (i, 0)),
                pl.BlockSpec((NC, TB, D), lambda i: (0, i, 0)),
                pl.BlockSpec((D, D), lambda i: (0, 0)),
                pl.BlockSpec((D, D), lambda i: (0, 0)),
                pl.BlockSpec((D, D), lambda i: (0, 0)),
                pl.BlockSpec((1, D), lambda i: (0, 0)),
            ],
            out_specs=[
                pl.BlockSpec((TB, D), lambda i: (i, 0)),
                pl.BlockSpec((1, D), lambda i: (0, 0)),
            ],
            out_shape=[
                jax.ShapeDtypeStruct((N, D), F32),
                jax.ShapeDtypeStruct((1, D), F32),
            ],
        )(x, s1, p2, w0, w1, w2, b)

    return run


_tag_act = _make_tag(True)
_tag_lin = _make_tag(False)


def kernel(x, edge_index, edge_weight, rep0_lin1_W, rep0_lin1_b, rep0_lin2_W,
           rep0_lin2_b, rep0_ln_w, rep0_ln_b, tag0_W, tag0_b, rep1_lin1_W,
           rep1_lin1_b, rep1_lin2_W, rep1_lin2_b, rep1_ln_w, rep1_ln_b,
           tag1_W, tag1_b):
    rows = edge_index[0]
    cols = edge_index[1]
    pad = EPAD - E
    pad_idx = jnp.arange(pad, dtype=jnp.int32) % N
    rows_p = jnp.concatenate([rows, pad_idx]).reshape(NW, NCH, C)
    cols_p = jnp.concatenate([cols, pad_idx]).reshape(NW, NCH, C)
    w_p = jnp.concatenate([edge_weight,
                           jnp.zeros((pad,), F32)]).reshape(NW, NCH, C)
    w_win = w_p.reshape(NW, NCH * C // 128, 128)

    b1_0 = rep0_lin1_b.reshape(1, D)
    b2_0 = rep0_lin2_b.reshape(1, D)
    lnw0 = rep0_ln_w.reshape(1, D)
    lnb0 = rep0_ln_b.reshape(1, D)
    b1_1 = rep1_lin1_b.reshape(1, D)
    b2_1 = rep1_lin2_b.reshape(1, D)
    lnw1 = rep1_ln_w.reshape(1, D)
    lnb1 = rep1_ln_b.reshape(1, D)
    t0b = tag0_b.reshape(1, D)
    t1b = tag1_b.reshape(1, D)
    t0w = [tag0_W[:, :D], tag0_W[:, D:2 * D], tag0_W[:, 2 * D:]]
    t1w = [tag1_W[:, :D], tag1_W[:, D:2 * D], tag1_W[:, 2 * D:]]

    slp = _sc_sumlines(cols_p, w_p)
    slp3 = slp[:, :N].reshape(NC, N, 1)
    sumlines, colsum0, wsum = _prep(slp3, x)

    y0, avg0 = _repy(colsum0, wsum, rep0_lin1_W, b1_0, rep0_lin2_W, b2_0,
                     lnw0, lnb0)
    x1 = _repx(x, y0)
    k0 = _repcov(x, sumlines, wsum, avg0, y0, jnp.zeros((1, 1), F32))

    p1 = _sc_spmm(x1, rows_p, cols_p, w_win)
    s1 = _addp(p1)
    p2 = _sc_spmm(s1, rows_p, cols_p, w_win)
    x2, colsum1 = _tag_act(x1, s1, p2, t0w[0], t0w[1], t0w[2], t0b)

    y1, avg1 = _repy(colsum1, wsum, rep1_lin1_W, b1_1, rep1_lin2_W, b2_1,
                     lnw1, lnb1)
    x3 = _repx(x2, y1)
    k1 = _repcov(x2, sumlines, wsum, avg1, y1, k0)

    p3 = _sc_spmm(x3, rows_p, cols_p, w_win)
    s1b = _addp(p3)
    p4 = _sc_spmm(s1b, rows_p, cols_p, w_win)
    x4, _ = _tag_lin(x3, s1b, p4, t1w[0], t1w[1], t1w[2], t1b)

    return (x4, k1[0, 0])


# speedup vs baseline: 1.0317x; 1.0317x over previous
import functools

import jax
import jax.numpy as jnp
from jax import lax
from jax.experimental import pallas as pl
from jax.experimental.pallas import tpu as pltpu
from jax.experimental.pallas import tpu_sc as plsc

N = 10000
E = 320000
D = 128
NC = 2
NS = 16
NW = NC * NS
C = 64
NCH = 160
IW = 32
NWIN = NCH // IW
EPAD = NW * NCH * C
NPAD = 10240
RPT = NPAD // NS
TB = 1000
GRID = N // TB

F32 = jnp.float32
_mesh = plsc.VectorSubcoreMesh(core_axis_name="c", subcore_axis_name="s",
                               num_cores=NC, num_subcores=NS)
_HIGH = jax.lax.Precision.HIGHEST


def _dot_t(a, b):
    return lax.dot_general(a, b, (((1,), (1,)), ((), ())),
                           precision=_HIGH, preferred_element_type=F32)


def _dot_cc0(a, b):
    return lax.dot_general(a, b, (((0,), (0,)), ((), ())),
                           precision=_HIGH, preferred_element_type=F32)


@functools.partial(
    pl.kernel,
    out_type=jax.ShapeDtypeStruct((NC, NPAD), F32),
    mesh=_mesh,
    scratch_types=[
        pltpu.VMEM((NCH, C), jnp.int32),
        pltpu.VMEM((NCH, C), F32),
        pltpu.VMEM((RPT,), F32),
        pltpu.VMEM_SHARED((NPAD,), F32),
        pltpu.SemaphoreType.DMA,
    ],
)
def _sc_sumlines(cols_hbm, w_hbm, out_hbm, cols_v, w_v, zb, acc, ssem):
    c = lax.axis_index("c")
    s = lax.axis_index("s")
    wid = c * NS + s
    for i in range(RPT // 16):
        zb[pl.ds(i * 16, 16)] = jnp.zeros((16,), F32)
    pltpu.sync_copy(zb, acc.at[pl.ds(s * RPT, RPT)])
    pltpu.sync_copy(cols_hbm.at[wid], cols_v)
    pltpu.sync_copy(w_hbm.at[wid], w_v)
    plsc.subcore_barrier()

    def fire(j, carry):
        pltpu.sync_copy(w_v.at[j], acc.at[cols_v.at[j]], add=True)
        return carry

    lax.fori_loop(0, NCH, fire, 0)
    plsc.subcore_barrier()
    pltpu.sync_copy(acc.at[pl.ds(s * RPT, RPT)],
                    out_hbm.at[c, pl.ds(s * RPT, RPT)])


def _make_sc_spmm():
    @functools.partial(
        pl.kernel,
        out_type=jax.ShapeDtypeStruct((NC, NPAD, D), F32),
        mesh=_mesh,
        scratch_types=[
            pltpu.VMEM((IW, C), jnp.int32),
            pltpu.VMEM((IW, C), jnp.int32),
            pltpu.VMEM((IW // 2, 128), F32),
            pltpu.VMEM((C, D), F32),
            pltpu.VMEM((C, D), F32),
            pltpu.VMEM((C, D), F32),
            pltpu.VMEM((C, D), F32),
            pltpu.VMEM_SHARED((NPAD, D), F32),
            pltpu.SemaphoreType.DMA,
            pltpu.SemaphoreType.DMA,
            pltpu.SemaphoreType.DMA,
            pltpu.SemaphoreType.DMA,
            pltpu.SemaphoreType.DMA,
            pltpu.SemaphoreType.DMA,
            pltpu.SemaphoreType.DMA,
            pltpu.SemaphoreType.DMA,
        ],
    )
    def spmm(x_hbm, rows_hbm, cols_hbm, w_hbm, out_hbm,
             rw, cw, ww, gb, gb1, gb2, gb3, acc,
             gsem, gsem1, gsem2, gsem3, ssem, ssem1, ssem2, ssem3):
        c = lax.axis_index("c")
        s = lax.axis_index("s")
        wid = c * NS + s

        def zrow(r, carry):
            for v in range(8):
                gb[r, pl.ds(v * 16, 16)] = jnp.zeros((16,), F32)
            return carry

        lax.fori_loop(0, C, zrow, 0)
        for t in range(RPT // C):
            pltpu.sync_copy(gb, acc.at[pl.ds(s * RPT + t * C, C)])
        plsc.subcore_barrier()

        gbs = [gb, gb1, gb2, gb3]
        gss = [gsem, gsem1, gsem2, gsem3]
        sss = [ssem, ssem1, ssem2, ssem3]

        def win_body(w):
            pltpu.sync_copy(rows_hbm.at[wid, pl.ds(w * IW, IW)], rw)
            pltpu.sync_copy(cols_hbm.at[wid, pl.ds(w * IW, IW)], cw)
            wrows = IW * C // 128
            pltpu.sync_copy(w_hbm.at[wid, pl.ds(w * wrows, wrows)], ww)
            pltpu.async_copy(x_hbm.at[cw.at[0]], gbs[0], gss[0])

            def quad(t4, carry):
                for u in range(4):
                    t = t4 * 4 + u
                    b = u
                    bp = (u + 2) % 4
                    bn = (u + 1) % 4

                    @pl.when(t >= 2)
                    def _(t=t, bp=bp):
                        pltpu.make_async_copy(gbs[bp], acc.at[rw.at[t - 2]],
                                              sss[bp]).wait()

                    @pl.when(t + 1 < IW)
                    def _(t=t, bn=bn):
                        pltpu.async_copy(x_hbm.at[cw.at[t + 1]], gbs[bn],
                                         gss[bn])

                    pltpu.make_async_copy(x_hbm.at[cw.at[t]], gbs[b],
                                          gss[b]).wait()
                    th = t4 * 2 + u // 2

                    def sgrp(g, carry2, b=b, th=th, off=(u % 2) * C):
                        wvec = ww[th, pl.ds(off + g * 16, 16)]
                        for l in range(16):
                            wv = jnp.full((16,), wvec[l], F32)
                            e = g * 16 + l
                            for v in range(8):
                                sl = pl.ds(v * 16, 16)
                                gbs[b][e, sl] = gbs[b][e, sl] * wv
                        return carry2

                    lax.fori_loop(0, C // 16, sgrp, 0)

                    @pl.when(t < IW)
                    def _(t=t, b=b):
                        pltpu.async_copy(gbs[b], acc.at[rw.at[t]], sss[b],
                                         add=True)
                return carry

            lax.fori_loop(0, IW // 4, quad, 0)
            for t in (IW - 2, IW - 1):
                pltpu.make_async_copy(gbs[t % 4], acc.at[rw.at[t]],
                                      sss[t % 4]).wait()

        for w in range(NWIN):
            win_body(w)
        plsc.subcore_barrier()
        pltpu.sync_copy(acc.at[pl.ds(s * RPT, RPT)],
                        out_hbm.at[c, pl.ds(s * RPT, RPT)])

    return spmm


_sc_spmm = _make_sc_spmm()


def _prep_body(slp_ref, x_ref, sl_ref, cs_ref, ws_ref):
    i = pl.program_id(0)
    sl = slp_ref[0] + slp_ref[1]
    sl_ref[...] = sl

    @pl.when(i == 0)
    def _():
        cs_ref[...] = jnp.zeros((1, D), F32)
        ws_ref[...] = jnp.zeros((1, 1), F32)

    cs_ref[...] += jnp.sum(x_ref[...], axis=0, keepdims=True)
    ws_ref[...] += jnp.sum(sl).reshape(1, 1)


def _prep(slp, x):
    return pl.pallas_call(
        _prep_body,
        grid=(GRID,),
        in_specs=[
            pl.BlockSpec((NC, TB, 1), lambda i: (0, i, 0)),
            pl.BlockSpec((TB, D), lambda i: (i, 0)),
        ],
        out_specs=[
            pl.BlockSpec((TB, 1), lambda i: (i, 0)),
            pl.BlockSpec((1, D), lambda i: (0, 0)),
            pl.BlockSpec((1, 1), lambda i: (0, 0)),
        ],
        out_shape=[
            jax.ShapeDtypeStruct((N, 1), F32),
            jax.ShapeDtypeStruct((1, D), F32),
            jax.ShapeDtypeStruct((1, 1), F32),
        ],
    )(slp, x)


def _repy_body(cs_ref, ws_ref, w1_ref, b1_ref, w2_ref, b2_ref,
               lnw_ref, lnb_ref, y_ref, avg_ref):
    wsum = ws_ref[0, 0]
    avg = cs_ref[...] / wsum
    t = _dot_t(avg, w1_ref[...]) + b1_ref[...]
    t = jnp.maximum(t, 0.0)
    mu = jnp.mean(t, axis=1, keepdims=True)
    var = jnp.mean((t - mu) ** 2, axis=1, keepdims=True)
    t = (t - mu) / jnp.sqrt(var + 1e-5) * lnw_ref[...] + lnb_ref[...]
    t = _dot_t(t, w2_ref[...]) + b2_ref[...]
    y_ref[...] = 0.25 + 2.0 * jax.nn.sigmoid(t)
    avg_ref[...] = avg


def _repy(cs, ws, w1, b1, w2, b2, lnw, lnb):
    spec1d = pl.BlockSpec((1, D), lambda: (0, 0))
    return pl.pallas_call(
        _repy_body,
        in_specs=[spec1d, pl.BlockSpec((1, 1), lambda: (0, 0)),
                  pl.BlockSpec((D, D), lambda: (0, 0)), spec1d,
                  pl.BlockSpec((D, D), lambda: (0, 0)), spec1d,
                  spec1d, spec1d],
        out_specs=[spec1d, spec1d],
        out_shape=[jax.ShapeDtypeStruct((1, D), F32),
                   jax.ShapeDtypeStruct((1, D), F32)],
    )(cs, ws, w1, b1, w2, b2, lnw, lnb)


def _repx_body(x_ref, y_ref, xo_ref):
    xo_ref[...] = x_ref[...] * y_ref[...]


def _repx(x, y):
    return pl.pallas_call(
        _repx_body,
        grid=(GRID,),
        in_specs=[pl.BlockSpec((TB, D), lambda i: (i, 0)),
                  pl.BlockSpec((1, D), lambda i: (0, 0))],
        out_specs=pl.BlockSpec((TB, D), lambda i: (i, 0)),
        out_shape=jax.ShapeDtypeStruct((NPAD, D), F32),
    )(x, y)


def _repcov_body(x_ref, sl_ref, ws_ref, avg_ref, y_ref, kprev_ref,
                 k_ref, cov_s):
    i = pl.program_id(0)
    wsum = ws_ref[0, 0]

    @pl.when(i == 0)
    def _():
        cov_s[...] = jnp.zeros((D, D), F32)

    xb = x_ref[...]
    slb = sl_ref[...]
    d = jnp.sqrt(slb / wsum) * (xb / slb - avg_ref[...])
    cov_s[...] += _dot_cc0(d, d)

    @pl.when(i == GRID - 1)
    def _():
        cov = cov_s[...]
        q = y_ref[...]
        Q = _dot_cc0(q, q)
        eye = (lax.broadcasted_iota(jnp.int32, (D, D), 0)
               == lax.broadcasted_iota(jnp.int32, (D, D), 1)).astype(F32)
        t1 = jnp.sum(eye * cov * Q)
        t2 = jnp.sum(cov * Q)
        loss = t1 - (t2 - t1) / D
        c1 = jnp.sum(eye * cov)
        c2 = jnp.sum(cov)
        lscov = c1 - (c2 - c1) / D
        qq = jnp.sum(q * q)
        k_ref[...] = (kprev_ref[0, 0]
                      + loss / lscov * jnp.float32(D) / qq).reshape(1, 1)


def _repcov(x, sl, ws, avg, y, kprev):
    return pl.pallas_call(
        _repcov_body,
        grid=(GRID,),
        in_specs=[
            pl.BlockSpec((TB, D), lambda i: (i, 0)),
            pl.BlockSpec((TB, 1), lambda i: (i, 0)),
            pl.BlockSpec((1, 1), lambda i: (0, 0)),
            pl.BlockSpec((1, D), lambda i: (0, 0)),
            pl.BlockSpec((1, D), lambda i: (0, 0)),
            pl.BlockSpec((1, 1), lambda i: (0, 0)),
        ],
        out_specs=[pl.BlockSpec((1, 1), lambda i: (0, 0))],
        out_shape=[jax.ShapeDtypeStruct((1, 1), F32)],
        scratch_shapes=[pltpu.VMEM((D, D), F32)],
    )(x, sl, ws, avg, y, kprev)[0]


def _addp_body(p_ref, o_ref):
    o_ref[...] = p_ref[0] + p_ref[1]


def _addp(p):
    blk = NPAD // GRID
    return pl.pallas_call(
        _addp_body,
        grid=(GRID,),
        in_specs=[pl.BlockSpec((NC, blk, D), lambda i: (0, i, 0))],
        out_specs=pl.BlockSpec((blk, D), lambda i: (i, 0)),
        out_shape=jax.ShapeDtypeStruct((NPAD, D), F32),
    )(p)


def _make_tag(act):
    def body(x_ref, s1_ref, p2_ref, w0_ref, w1_ref, w2_ref, b_ref,
             xo_ref, cs_ref):
        i = pl.program_id(0)
        s2 = p2_ref[0] + p2_ref[1]
        o = (_dot_t(x_ref[...], w0_ref[...])
             + _dot_t(s1_ref[...], w1_ref[...])
             + _dot_t(s2, w2_ref[...])
             + b_ref[...])
        if act:
            o = jnp.where(o >= 0, o, 0.01 * o)
        xo_ref[...] = o

        @pl.when(i == 0)
        def _():
            cs_ref[...] = jnp.zeros((1, D), F32)

        cs_ref[...] += jnp.sum(o, axis=0, keepdims=True)

    def run(x, s1, p2, w0, w1, w2, b):
        return pl.pallas_call(
            body,
            grid=(GRID,),
            in_specs=[
                pl.BlockSpec((TB, D), lambda i: (i, 0)),
                pl.BlockSpec((TB, D), lambda i: (i, 0)),
                pl.BlockSpec((NC, TB, D), lambda i: (0, i, 0)),
                pl.BlockSpec((D, D), lambda i: (0, 0)),
                pl.BlockSpec((D, D), lambda i: (0, 0)),
                pl.BlockSpec((D, D), lambda i: (0, 0)),
                pl.BlockSpec((1, D), lambda i: (0, 0)),
            ],
            out_specs=[
                pl.BlockSpec((TB, D), lambda i: (i, 0)),
                pl.BlockSpec((1, D), lambda i: (0, 0)),
            ],
            out_shape=[
                jax.ShapeDtypeStruct((N, D), F32),
                jax.ShapeDtypeStruct((1, D), F32),
            ],
        )(x, s1, p2, w0, w1, w2, b)

    return run


_tag_act = _make_tag(True)
_tag_lin = _make_tag(False)


def kernel(x, edge_index, edge_weight, rep0_lin1_W, rep0_lin1_b, rep0_lin2_W,
           rep0_lin2_b, rep0_ln_w, rep0_ln_b, tag0_W, tag0_b, rep1_lin1_W,
           rep1_lin1_b, rep1_lin2_W, rep1_lin2_b, rep1_ln_w, rep1_ln_b,
           tag1_W, tag1_b):
    rows = edge_index[0]
    cols = edge_index[1]
    pad = EPAD - E
    pad_idx = jnp.arange(pad, dtype=jnp.int32) % N
    rows_p = jnp.concatenate([rows, pad_idx]).reshape(NW, NCH, C)
    cols_p = jnp.concatenate([cols, pad_idx]).reshape(NW, NCH, C)
    w_p = jnp.concatenate([edge_weight,
                           jnp.zeros((pad,), F32)]).reshape(NW, NCH, C)
    w_win = w_p.reshape(NW, NCH * C // 128, 128)

    b1_0 = rep0_lin1_b.reshape(1, D)
    b2_0 = rep0_lin2_b.reshape(1, D)
    lnw0 = rep0_ln_w.reshape(1, D)
    lnb0 = rep0_ln_b.reshape(1, D)
    b1_1 = rep1_lin1_b.reshape(1, D)
    b2_1 = rep1_lin2_b.reshape(1, D)
    lnw1 = rep1_ln_w.reshape(1, D)
    lnb1 = rep1_ln_b.reshape(1, D)
    t0b = tag0_b.reshape(1, D)
    t1b = tag1_b.reshape(1, D)
    t0w = [tag0_W[:, :D], tag0_W[:, D:2 * D], tag0_W[:, 2 * D:]]
    t1w = [tag1_W[:, :D], tag1_W[:, D:2 * D], tag1_W[:, 2 * D:]]

    slp = _sc_sumlines(cols_p, w_p)
    slp3 = slp[:, :N].reshape(NC, N, 1)
    sumlines, colsum0, wsum = _prep(slp3, x)

    y0, avg0 = _repy(colsum0, wsum, rep0_lin1_W, b1_0, rep0_lin2_W, b2_0,
                     lnw0, lnb0)
    x1 = _repx(x, y0)
    k0 = _repcov(x, sumlines, wsum, avg0, y0, jnp.zeros((1, 1), F32))

    p1 = _sc_spmm(x1, rows_p, cols_p, w_win)
    s1 = _addp(p1)
    p2 = _sc_spmm(s1, rows_p, cols_p, w_win)
    x2, colsum1 = _tag_act(x1, s1, p2, t0w[0], t0w[1], t0w[2], t0b)

    y1, avg1 = _repy(colsum1, wsum, rep1_lin1_W, b1_1, rep1_lin2_W, b2_1,
                     lnw1, lnb1)
    x3 = _repx(x2, y1)
    k1 = _repcov(x2, sumlines, wsum, avg1, y1, k0)

    p3 = _sc_spmm(x3, rows_p, cols_p, w_win)
    s1b = _addp(p3)
    p4 = _sc_spmm(s1b, rows_p, cols_p, w_win)
    x4, _ = _tag_lin(x3, s1b, p4, t1w[0], t1w[1], t1w[2], t1b)

    return (x4, k1[0, 0])
